# Initial kernel scaffold; baseline (speedup 1.0000x reference)
import jax
import jax.numpy as jnp
from jax.experimental import pallas as pl


def kernel(x, edge_index, W1, b1, W2, b2):
    raise NotImplementedError("write your pallas kernel here")


# speedup vs baseline: 20.4181x; 20.4181x over previous
import functools

import jax
import jax.numpy as jnp
from jax import lax
from jax.experimental import pallas as pl
from jax.experimental.pallas import tpu as pltpu
from jax.experimental.pallas import tpu_sc as plsc

NC = 2
NS = 16
NW = NC * NS
LANES = 16

N = 10000
NP = 10240
D_IN = 128
H = 256
D_OUT = 2
D2 = 16

EDGE_BATCH = 128
ROWS_PER_TILE = NP // NS

_mesh = plsc.VectorSubcoreMesh(
    core_axis_name="c", subcore_axis_name="s", num_cores=NC, num_subcores=NS)


def _nbatch(e_total):
    return (e_total + NW * EDGE_BATCH - 1) // (NW * EDGE_BATCH)


def _make_deg(nb):
    @functools.partial(
        pl.kernel, mesh=_mesh,
        out_type=jax.ShapeDtypeStruct((NC, NP), jnp.float32),
        scratch_types=[
            pltpu.VMEM((nb, EDGE_BATCH), jnp.int32),
            pltpu.VMEM((ROWS_PER_TILE,), jnp.float32),
            pltpu.VMEM((EDGE_BATCH,), jnp.float32),
            pltpu.VMEM_SHARED((NP,), jnp.float32),
            pltpu.SemaphoreType.DMA,
        ],
    )
    def deg_kernel(dstp_hbm, out_hbm, dstv, zv, onesv, shared, sem):
        c = lax.axis_index("c")
        s = lax.axis_index("s")
        for i in range(ROWS_PER_TILE // LANES):
            zv[pl.ds(i * LANES, LANES)] = jnp.zeros((LANES,), jnp.float32)
        for i in range(EDGE_BATCH // LANES):
            onesv[pl.ds(i * LANES, LANES)] = jnp.ones((LANES,), jnp.float32)
        pltpu.sync_copy(dstp_hbm.at[c, s], dstv)

        pltpu.sync_copy(zv, shared.at[pl.ds(s * ROWS_PER_TILE, ROWS_PER_TILE)])
        plsc.subcore_barrier()

        def body(j, carry):
            pltpu.sync_copy(onesv, shared.at[dstv.at[j]], add=True)
            return carry
        lax.fori_loop(0, nb, body, 0)
        plsc.subcore_barrier()
        pltpu.sync_copy(
            shared.at[pl.ds(s * ROWS_PER_TILE, ROWS_PER_TILE)],
            out_hbm.at[c, pl.ds(s * ROWS_PER_TILE, ROWS_PER_TILE)])

    return deg_kernel


def _make_agg(d, nb):
    @functools.partial(
        pl.kernel, mesh=_mesh,
        out_type=jax.ShapeDtypeStruct((NC, NP, d), jnp.float32),
        scratch_types=[
            pltpu.VMEM((nb, EDGE_BATCH), jnp.int32),
            pltpu.VMEM((nb, EDGE_BATCH), jnp.int32),
            pltpu.VMEM((EDGE_BATCH, d), jnp.float32),
            pltpu.VMEM_SHARED((NP, d), jnp.float32),
            pltpu.SemaphoreType.DMA,
        ],
    )
    def agg_kernel(u_hbm, srcp_hbm, dstp_hbm, out_hbm, srcv, dstv, rows, shared, sem):
        c = lax.axis_index("c")
        s = lax.axis_index("s")
        pltpu.sync_copy(srcp_hbm.at[c, s], srcv)
        pltpu.sync_copy(dstp_hbm.at[c, s], dstv)
        for r in range(EDGE_BATCH):
            for k in range(d // LANES):
                rows[r, pl.ds(k * LANES, LANES)] = jnp.zeros((LANES,), jnp.float32)

        for k in range(ROWS_PER_TILE // EDGE_BATCH):
            pltpu.sync_copy(
                rows,
                shared.at[pl.ds(s * ROWS_PER_TILE + k * EDGE_BATCH, EDGE_BATCH)])
        plsc.subcore_barrier()

        def body(j, carry):
            pltpu.async_copy(u_hbm.at[srcv.at[j]], rows, sem).wait()
            pltpu.sync_copy(rows, shared.at[dstv.at[j]], add=True)
            return carry
        lax.fori_loop(0, nb, body, 0)
        plsc.subcore_barrier()
        pltpu.sync_copy(
            shared.at[pl.ds(s * ROWS_PER_TILE, ROWS_PER_TILE)],
            out_hbm.at[c, pl.ds(s * ROWS_PER_TILE, ROWS_PER_TILE)])

    return agg_kernel


def _make_agg2(nb):
    np2 = 2 * NP
    sl = np2 // NS

    @functools.partial(
        pl.kernel, mesh=_mesh,
        out_type=jax.ShapeDtypeStruct((NC, np2), jnp.float32),
        scratch_types=[
            pltpu.VMEM((nb, EDGE_BATCH), jnp.int32),
            pltpu.VMEM((nb, EDGE_BATCH), jnp.int32),
            pltpu.VMEM((nb, EDGE_BATCH), jnp.int32),
            pltpu.VMEM((nb, EDGE_BATCH), jnp.int32),
            pltpu.VMEM((EDGE_BATCH,), jnp.float32),
            pltpu.VMEM_SHARED((np2,), jnp.float32),
            pltpu.SemaphoreType.DMA,
        ],
    )
    def agg2_kernel(z_hbm, srcp0_hbm, srcp1_hbm, dstp0_hbm, dstp1_hbm, out_hbm,
                    srcv0, srcv1, dstv0, dstv1, g, shared, sem):
        c = lax.axis_index("c")
        s = lax.axis_index("s")
        pltpu.sync_copy(srcp0_hbm.at[c, s], srcv0)
        pltpu.sync_copy(srcp1_hbm.at[c, s], srcv1)
        pltpu.sync_copy(dstp0_hbm.at[c, s], dstv0)
        pltpu.sync_copy(dstp1_hbm.at[c, s], dstv1)
        for i in range(EDGE_BATCH // LANES):
            g[pl.ds(i * LANES, LANES)] = jnp.zeros((LANES,), jnp.float32)
        for k in range(sl // EDGE_BATCH):
            pltpu.sync_copy(g, shared.at[pl.ds(s * sl + k * EDGE_BATCH, EDGE_BATCH)])
        plsc.subcore_barrier()

        def body(j, carry):
            pltpu.async_copy(z_hbm.at[srcv0.at[j]], g, sem).wait()
            pltpu.sync_copy(g, shared.at[dstv0.at[j]], add=True)
            pltpu.async_copy(z_hbm.at[srcv1.at[j]], g, sem).wait()
            pltpu.sync_copy(g, shared.at[dstv1.at[j]], add=True)
            return carry
        lax.fori_loop(0, nb, body, 0)
        plsc.subcore_barrier()
        pltpu.sync_copy(shared.at[pl.ds(s * sl, sl)],
                        out_hbm.at[c, pl.ds(s * sl, sl)])

    return agg2_kernel


def _prep_body(degt_ref, x_ref, u_ref, dinv_ref):
    deg = degt_ref[:, 0:1] + degt_ref[:, 1:2] + 1.0
    dinv = lax.rsqrt(deg)
    dinv_ref[...] = dinv
    u_ref[...] = x_ref[...] * dinv


def _l1_body(p_ref, u_ref, dinv_ref, w1_ref, b1_ref, w2_ref, u2_ref):
    dinv = dinv_ref[...]
    y = (p_ref[0] + p_ref[1] + u_ref[...]) * dinv
    h = jnp.maximum(
        jnp.dot(y, w1_ref[...], preferred_element_type=jnp.float32) + b1_ref[...],
        0.0)
    u2_ref[...] = jnp.dot(h, w2_ref[...], preferred_element_type=jnp.float32) * dinv


def _l2_body(q_ref, zt_ref, dinvt_ref, b2_ref, o_ref):
    q = (q_ref[0] + q_ref[1] + zt_ref[...]) * dinvt_ref[...]
    o_ref[...] = jnp.tanh(q + b2_ref[...])


def kernel(x, edge_index, W1, b1, W2, b2):
    e_total = edge_index.shape[1]
    nb = _nbatch(e_total)
    ew = nb * EDGE_BATCH
    padlen = NW * ew - e_total

    ei = edge_index.astype(jnp.int32)
    pad = jnp.full((padlen,), N, jnp.int32)
    srcp = jnp.concatenate([ei[0], pad]).reshape(NC, NS, nb, EDGE_BATCH)
    dstp = jnp.concatenate([ei[1], pad]).reshape(NC, NS, nb, EDGE_BATCH)
    xp = jnp.pad(x, ((0, NP - N), (0, 0)))

    deg_p = _make_deg(nb)(dstp)
    deg_t = deg_p.T

    u, dinv = pl.pallas_call(
        _prep_body,
        out_shape=[jax.ShapeDtypeStruct((NP, D_IN), jnp.float32),
                   jax.ShapeDtypeStruct((NP, 1), jnp.float32)],
    )(deg_t, xp)

    p_part = _make_agg(D_IN, nb)(u, srcp, dstp)

    w2p = jnp.pad(W2, ((0, 0), (0, D2 - D_OUT)))
    bm = 1280
    grid = NP // bm
    u2 = pl.pallas_call(
        _l1_body,
        grid=(grid,),
        in_specs=[
            pl.BlockSpec((NC, bm, D_IN), lambda i: (0, i, 0)),
            pl.BlockSpec((bm, D_IN), lambda i: (i, 0)),
            pl.BlockSpec((bm, 1), lambda i: (i, 0)),
            pl.BlockSpec((D_IN, H), lambda i: (0, 0)),
            pl.BlockSpec((1, H), lambda i: (0, 0)),
            pl.BlockSpec((H, D2), lambda i: (0, 0)),
        ],
        out_specs=pl.BlockSpec((bm, D2), lambda i: (i, 0)),
        out_shape=jax.ShapeDtypeStruct((NP, D2), jnp.float32),
    )(p_part, u, dinv, W1, b1.reshape(1, H), w2p)

    zt = u2[:, :D_OUT].T
    zflat = zt.reshape(2 * NP)
    q_flat = _make_agg2(nb)(zflat, srcp, srcp + NP, dstp, dstp + NP)
    q_part = q_flat.reshape(NC, D_OUT, NP)

    out_t = pl.pallas_call(
        _l2_body,
        out_shape=jax.ShapeDtypeStruct((D_OUT, NP), jnp.float32),
    )(q_part, zt, dinv.T, b2.reshape(D_OUT, 1))

    return out_t.T[:N]
